# initial kernel scaffold (unmeasured)
import functools

import jax
import jax.numpy as jnp
from jax import lax
from jax.experimental import pallas as pl
from jax.experimental.pallas import tpu as pltpu

B, S, H, Dh, Dr = 4, 256, 32, 128, 64
D = 4096
DC = 512
Y = 4
DC_SH = DC // Y
BS = B * S

_CompilerParams = getattr(pltpu, "CompilerParams", None) or getattr(
    pltpu, "TPUCompilerParams"
)


def _allgather_body(x_ref, wdkv_ref, wuk_ref, wuv_ref, wkr_ref,
                    c_ref, uk_ref, uv_ref, kr_ref,
                    send_sems, recv_sems):
    my_x = lax.axis_index("x")
    my_y = lax.axis_index("y")
    my_z = lax.axis_index("z")
    left = (my_y - 1) % Y
    right = (my_y + 1) % Y

    barrier_sem = pltpu.get_barrier_semaphore()
    for nbr in (left, right):
        pl.semaphore_signal(
            barrier_sem, inc=1,
            device_id=(my_x, nbr, my_z),
            device_id_type=pl.DeviceIdType.MESH,
        )
    pl.semaphore_wait(barrier_sem, 2)

    x = x_ref[...]
    c_local = jnp.dot(x, wdkv_ref[...], preferred_element_type=jnp.float32)
    kr_ref[...] = jnp.dot(x, wkr_ref[...], preferred_element_type=jnp.float32)

    c_ref[my_y] = c_local
    uk_ref[my_y] = wuk_ref[...]
    uv_ref[my_y] = wuv_ref[...]

    for h in range(Y - 1):
        origin = (my_y - h) % Y
        rdmas = []
        for t, buf in enumerate((c_ref, uk_ref, uv_ref)):
            rdma = pltpu.make_async_remote_copy(
                src_ref=buf.at[origin],
                dst_ref=buf.at[origin],
                send_sem=send_sems.at[t, h],
                recv_sem=recv_sems.at[t, h],
                device_id=(my_x, right, my_z),
                device_id_type=pl.DeviceIdType.MESH,
            )
            rdma.start()
            rdmas.append(rdma)
        for rdma in rdmas:
            rdma.wait()


def _mla_body(x_ref, c_ref, uk_ref, uv_ref, kr_ref,
              wq_ref, wqr_ref, wo_ref, out_ref, o_scratch):
    h = pl.program_id(0)
    x = x_ref[...]

    q = jnp.dot(x, wq_ref[...], preferred_element_type=jnp.float32)
    qr = jnp.dot(x, wqr_ref[...], preferred_element_type=jnp.float32)
    k = jnp.zeros((BS, Dh), jnp.float32)
    v = jnp.zeros((BS, Dh), jnp.float32)
    for yi in range(Y):
        k = k + jnp.dot(c_ref[yi], uk_ref[yi],
                        preferred_element_type=jnp.float32)
        v = v + jnp.dot(c_ref[yi], uv_ref[yi],
                        preferred_element_type=jnp.float32)
    kr = kr_ref[...]

    scale = (Dh + Dr) ** -0.5
    nt_dims = (((1,), (1,)), ((), ()))
    for b in range(B):
        sl = slice(b * S, (b + 1) * S)
        qb, qrb, kb, vb, krb = q[sl], qr[sl], k[sl], v[sl], kr[sl]
        s = lax.dot_general(qb, kb, nt_dims,
                            preferred_element_type=jnp.float32)
        s = s + lax.dot_general(qrb, krb, nt_dims,
                                preferred_element_type=jnp.float32)
        s = s * scale
        m = jnp.max(s, axis=-1, keepdims=True)
        p = jnp.exp(s - m)
        p = p / jnp.sum(p, axis=-1, keepdims=True)
        o_scratch[sl, :] = jnp.dot(p, vb, preferred_element_type=jnp.float32)

    contrib = jnp.dot(o_scratch[...], wo_ref[...],
                      preferred_element_type=jnp.float32)

    @pl.when(h == 0)
    def _():
        out_ref[...] = jnp.zeros_like(out_ref)

    out_ref[...] += contrib


def kernel(x, Wdkv, Wuk, Wuv, Wq, Wqr, Wkr, Wo):
    x2d = x.reshape(BS, D)

    c_all, uk_all, uv_all, kr = pl.pallas_call(
        _allgather_body,
        out_shape=(
            jax.ShapeDtypeStruct((Y, BS, DC_SH), jnp.float32),
            jax.ShapeDtypeStruct((Y, DC_SH, D), jnp.float32),
            jax.ShapeDtypeStruct((Y, DC_SH, D), jnp.float32),
            jax.ShapeDtypeStruct((BS, Dr), jnp.float32),
        ),
        in_specs=[pl.BlockSpec(memory_space=pltpu.VMEM)] * 5,
        out_specs=[pl.BlockSpec(memory_space=pltpu.VMEM)] * 4,
        scratch_shapes=[
            pltpu.SemaphoreType.DMA((3, Y - 1)),
            pltpu.SemaphoreType.DMA((3, Y - 1)),
        ],
        compiler_params=_CompilerParams(collective_id=0),
    )(x2d, Wdkv, Wuk, Wuv, Wkr)

    out2d = pl.pallas_call(
        _mla_body,
        grid=(H,),
        out_shape=jax.ShapeDtypeStruct((BS, D), jnp.float32),
        in_specs=[
            pl.BlockSpec((BS, D), lambda h: (0, 0)),
            pl.BlockSpec((Y, BS, DC_SH), lambda h: (0, 0, 0)),
            pl.BlockSpec((Y, DC_SH, Dh), lambda h: (0, 0, h)),
            pl.BlockSpec((Y, DC_SH, Dh), lambda h: (0, 0, h)),
            pl.BlockSpec((BS, Dr), lambda h: (0, 0)),
            pl.BlockSpec((D, Dh), lambda h: (0, h)),
            pl.BlockSpec((D, Dr), lambda h: (0, h)),
            pl.BlockSpec((Dh, D), lambda h: (h, 0)),
        ],
        out_specs=pl.BlockSpec((BS, D), lambda h: (0, 0)),
        scratch_shapes=[pltpu.VMEM((BS, Dh), jnp.float32)],
    )(x2d, c_all, uk_all, uv_all, kr, Wq, Wqr, Wo)

    return out2d.reshape(B, S, D)


# baseline (device time: 599767 ns/iter reference)
import functools

import jax
import jax.numpy as jnp
from jax import lax
from jax.experimental import pallas as pl
from jax.experimental.pallas import tpu as pltpu

B, S, H, Dh, Dr = 4, 256, 32, 128, 64
D = 4096
DC = 512
Y = 4
DC_SH = DC // Y
BS = B * S

_CompilerParams = getattr(pltpu, "CompilerParams", None) or getattr(
    pltpu, "TPUCompilerParams"
)

DEBUG_NO_RDMA = False
DEBUG_NO_BARRIER = False


def _allgather_body(x_ref, wdkv_ref, wuk_ref, wuv_ref, wkr_ref,
                    c_ref, uk_ref, uv_ref, kr_ref,
                    send_sems, recv_sems):
    my_x = lax.axis_index("x")
    my_y = lax.axis_index("y")
    my_z = lax.axis_index("z")

    x = x_ref[...]
    c_local = jnp.dot(x, wdkv_ref[...], preferred_element_type=jnp.float32)
    kr_ref[...] = jnp.dot(x, wkr_ref[...], preferred_element_type=jnp.float32)

    c_ref[my_y] = c_local
    uk_ref[my_y] = wuk_ref[...]
    uv_ref[my_y] = wuv_ref[...]

    if DEBUG_NO_RDMA:
        return

    right_id = (my_x, jnp.minimum(my_y + 1, Y - 1), my_z)
    left_id = (my_x, jnp.maximum(my_y - 1, 0), my_z)
    zero = jnp.int32(0)
    last = jnp.int32(Y - 1)

    bufs = (c_ref, uk_ref, uv_ref)
    for t in range(Y - 1):
        dirs = (
            (my_y - t, right_id, (my_y < Y - 1) & (my_y - t >= 0),
             my_y - 1 - t, (my_y > 0) & (my_y - 1 - t >= 0)),
            (my_y + t, left_id, (my_y > 0) & (my_y + t <= Y - 1),
             my_y + 1 + t, (my_y < Y - 1) & (my_y + 1 + t <= Y - 1)),
        )
        waits = []
        for d, (s_off, dev_id, s_cond, r_off, r_cond) in enumerate(dirs):
            s_slot = jnp.clip(s_off, zero, last)
            r_slot = jnp.clip(r_off, zero, last)
            for ti, buf in enumerate(bufs):
                send = pltpu.make_async_remote_copy(
                    src_ref=buf.at[s_slot],
                    dst_ref=buf.at[s_slot],
                    send_sem=send_sems.at[ti, t, d],
                    recv_sem=recv_sems.at[ti, t, d],
                    device_id=dev_id,
                    device_id_type=pl.DeviceIdType.MESH,
                )
                recv = pltpu.make_async_remote_copy(
                    src_ref=buf.at[r_slot],
                    dst_ref=buf.at[r_slot],
                    send_sem=send_sems.at[ti, t, d],
                    recv_sem=recv_sems.at[ti, t, d],
                    device_id=dev_id,
                    device_id_type=pl.DeviceIdType.MESH,
                )

                @pl.when(s_cond)
                def _(send=send):
                    send.start()

                waits.append((send, s_cond, recv, r_cond))

        for send, s_cond, recv, r_cond in waits:
            @pl.when(s_cond)
            def _(send=send):
                send.wait_send()

            @pl.when(r_cond)
            def _(recv=recv):
                recv.wait_recv()


def _mla_body(x_ref, c_ref, uk_ref, uv_ref, kr_ref,
              wq_ref, wqr_ref, wo_ref, out_ref, o_scratch):
    h = pl.program_id(0)
    x = x_ref[...]

    q = jnp.dot(x, wq_ref[...], preferred_element_type=jnp.float32)
    qr = jnp.dot(x, wqr_ref[0], preferred_element_type=jnp.float32)
    k = jnp.zeros((BS, Dh), jnp.float32)
    v = jnp.zeros((BS, Dh), jnp.float32)
    for yi in range(Y):
        k = k + jnp.dot(c_ref[yi], uk_ref[yi],
                        preferred_element_type=jnp.float32)
        v = v + jnp.dot(c_ref[yi], uv_ref[yi],
                        preferred_element_type=jnp.float32)
    kr = kr_ref[...]

    scale = (Dh + Dr) ** -0.5
    nt_dims = (((1,), (1,)), ((), ()))
    for b in range(B):
        sl = slice(b * S, (b + 1) * S)
        qb, qrb, kb, vb, krb = q[sl], qr[sl], k[sl], v[sl], kr[sl]
        s = lax.dot_general(qb, kb, nt_dims,
                            preferred_element_type=jnp.float32)
        s = s + lax.dot_general(qrb, krb, nt_dims,
                                preferred_element_type=jnp.float32)
        s = s * scale
        m = jnp.max(s, axis=-1, keepdims=True)
        p = jnp.exp(s - m)
        p = p / jnp.sum(p, axis=-1, keepdims=True)
        o_scratch[sl, :] = jnp.dot(p, vb, preferred_element_type=jnp.float32)

    contrib = jnp.dot(o_scratch[...], wo_ref[...],
                      preferred_element_type=jnp.float32)

    @pl.when(h == 0)
    def _():
        out_ref[...] = jnp.zeros_like(out_ref)

    out_ref[...] += contrib


def allgather_call(x2d, Wdkv, Wuk, Wuv, Wkr):
    return pl.pallas_call(
        _allgather_body,
        out_shape=(
            jax.ShapeDtypeStruct((Y, BS, DC_SH), jnp.float32),
            jax.ShapeDtypeStruct((Y, DC_SH, D), jnp.float32),
            jax.ShapeDtypeStruct((Y, DC_SH, D), jnp.float32),
            jax.ShapeDtypeStruct((BS, Dr), jnp.float32),
        ),
        in_specs=[pl.BlockSpec(memory_space=pltpu.VMEM)] * 5,
        out_specs=[pl.BlockSpec(memory_space=pltpu.VMEM)] * 4,
        scratch_shapes=[
            pltpu.SemaphoreType.DMA((3, Y - 1, 2)),
            pltpu.SemaphoreType.DMA((3, Y - 1, 2)),
        ],
        compiler_params=_CompilerParams(
            vmem_limit_bytes=100 * 1024 * 1024,
        ),
    )(x2d, Wdkv, Wuk, Wuv, Wkr)


def mla_call(x2d, c_all, uk_all, uv_all, kr, Wq, Wqr, Wo):
    return pl.pallas_call(
        _mla_body,
        grid=(H,),
        out_shape=jax.ShapeDtypeStruct((BS, D), jnp.float32),
        in_specs=[
            pl.BlockSpec((BS, D), lambda h: (0, 0)),
            pl.BlockSpec((Y, BS, DC_SH), lambda h: (0, 0, 0)),
            pl.BlockSpec((Y, DC_SH, Dh), lambda h: (0, 0, h)),
            pl.BlockSpec((Y, DC_SH, Dh), lambda h: (0, 0, h)),
            pl.BlockSpec((BS, Dr), lambda h: (0, 0)),
            pl.BlockSpec((D, Dh), lambda h: (0, h)),
            pl.BlockSpec((1, D, Dr), lambda h: (h, 0, 0)),
            pl.BlockSpec((Dh, D), lambda h: (h, 0)),
        ],
        out_specs=pl.BlockSpec((BS, D), lambda h: (0, 0)),
        scratch_shapes=[pltpu.VMEM((BS, Dh), jnp.float32)],
        compiler_params=_CompilerParams(
            vmem_limit_bytes=100 * 1024 * 1024
        ),
    )(x2d, c_all, uk_all, uv_all, kr, Wq,
      Wqr.reshape(D, H, Dr).transpose(1, 0, 2), Wo)


def kernel(x, Wdkv, Wuk, Wuv, Wq, Wqr, Wkr, Wo):
    x2d = x.reshape(BS, D)
    c_all, uk_all, uv_all, kr = allgather_call(x2d, Wdkv, Wuk, Wuv, Wkr)
    out2d = mla_call(x2d, c_all, uk_all, uv_all, kr, Wq, Wqr, Wo)
    return out2d.reshape(B, S, D)


# device time: 599665 ns/iter; 1.0002x vs baseline; 1.0002x over previous
import functools

import jax
import jax.numpy as jnp
from jax import lax
from jax.experimental import pallas as pl
from jax.experimental.pallas import tpu as pltpu

B, S, H, Dh, Dr = 4, 256, 32, 128, 64
D = 4096
DC = 512
Y = 4
DC_SH = DC // Y
BS = B * S

_CompilerParams = getattr(pltpu, "CompilerParams", None) or getattr(
    pltpu, "TPUCompilerParams"
)

DEBUG_NO_RDMA = False
DEBUG_NO_BARRIER = False


def _allgather_body(x_ref, wdkv_ref, wuk_ref, wuv_ref, wkr_ref,
                    c_ref, uk_ref, uv_ref, kr_ref,
                    send_sems, recv_sems):
    my_x = lax.axis_index("x")
    my_y = lax.axis_index("y")
    my_z = lax.axis_index("z")

    x = x_ref[...]
    c_local = jnp.dot(x, wdkv_ref[...], preferred_element_type=jnp.float32)
    kr_ref[...] = jnp.dot(x, wkr_ref[...], preferred_element_type=jnp.float32)

    c_ref[my_y] = c_local
    uk_ref[my_y] = wuk_ref[...]
    uv_ref[my_y] = wuv_ref[...]

    if DEBUG_NO_RDMA:
        return

    right_id = (my_x, jnp.minimum(my_y + 1, Y - 1), my_z)
    left_id = (my_x, jnp.maximum(my_y - 1, 0), my_z)
    zero = jnp.int32(0)
    last = jnp.int32(Y - 1)

    bufs = (c_ref, uk_ref, uv_ref)
    for t in range(Y - 1):
        dirs = (
            (my_y - t, right_id, (my_y < Y - 1) & (my_y - t >= 0),
             my_y - 1 - t, (my_y > 0) & (my_y - 1 - t >= 0)),
            (my_y + t, left_id, (my_y > 0) & (my_y + t <= Y - 1),
             my_y + 1 + t, (my_y < Y - 1) & (my_y + 1 + t <= Y - 1)),
        )
        waits = []
        for d, (s_off, dev_id, s_cond, r_off, r_cond) in enumerate(dirs):
            s_slot = jnp.clip(s_off, zero, last)
            r_slot = jnp.clip(r_off, zero, last)
            for ti, buf in enumerate(bufs):
                send = pltpu.make_async_remote_copy(
                    src_ref=buf.at[s_slot],
                    dst_ref=buf.at[s_slot],
                    send_sem=send_sems.at[ti, t, d],
                    recv_sem=recv_sems.at[ti, t, d],
                    device_id=dev_id,
                    device_id_type=pl.DeviceIdType.MESH,
                )
                recv = pltpu.make_async_remote_copy(
                    src_ref=buf.at[r_slot],
                    dst_ref=buf.at[r_slot],
                    send_sem=send_sems.at[ti, t, d],
                    recv_sem=recv_sems.at[ti, t, d],
                    device_id=dev_id,
                    device_id_type=pl.DeviceIdType.MESH,
                )

                @pl.when(s_cond)
                def _(send=send):
                    send.start()

                waits.append((send, s_cond, recv, r_cond))

        for send, s_cond, recv, r_cond in waits:
            @pl.when(s_cond)
            def _(send=send):
                send.wait_send()

            @pl.when(r_cond)
            def _(recv=recv):
                recv.wait_recv()


def _qr_body(x_ref, wqr_ref, qrt_ref):
    qr4 = jnp.dot(x_ref[...], wqr_ref[...],
                  preferred_element_type=jnp.float32)
    for i in range(4):
        qrt_ref[i] = qr4[:, i * Dr:(i + 1) * Dr]


def qr_call(x2d, Wqr):
    return pl.pallas_call(
        _qr_body,
        grid=(H // 4,),
        out_shape=jax.ShapeDtypeStruct((H, BS, Dr), jnp.float32),
        in_specs=[
            pl.BlockSpec((BS, D), lambda g: (0, 0)),
            pl.BlockSpec((D, 4 * Dr), lambda g: (0, g)),
        ],
        out_specs=pl.BlockSpec((4, BS, Dr), lambda g: (g, 0, 0)),
        compiler_params=_CompilerParams(
            vmem_limit_bytes=100 * 1024 * 1024,
        ),
    )(x2d, Wqr)


def _mla_body(x_ref, c_ref, uk_ref, uv_ref, kr_ref,
              wq_ref, wqr_ref, wo_ref, out_ref, o_scratch):
    h = pl.program_id(0)
    x = x_ref[...]

    q = jnp.dot(x, wq_ref[...], preferred_element_type=jnp.float32)
    qr = jnp.dot(x, wqr_ref[0], preferred_element_type=jnp.float32)
    k = jnp.zeros((BS, Dh), jnp.float32)
    v = jnp.zeros((BS, Dh), jnp.float32)
    for yi in range(Y):
        k = k + jnp.dot(c_ref[yi], uk_ref[yi],
                        preferred_element_type=jnp.float32)
        v = v + jnp.dot(c_ref[yi], uv_ref[yi],
                        preferred_element_type=jnp.float32)
    kr = kr_ref[...]

    scale = (Dh + Dr) ** -0.5
    nt_dims = (((1,), (1,)), ((), ()))
    for b in range(B):
        sl = slice(b * S, (b + 1) * S)
        qb, qrb, kb, vb, krb = q[sl], qr[sl], k[sl], v[sl], kr[sl]
        s = lax.dot_general(qb, kb, nt_dims,
                            preferred_element_type=jnp.float32)
        s = s + lax.dot_general(qrb, krb, nt_dims,
                                preferred_element_type=jnp.float32)
        s = s * scale
        m = jnp.max(s, axis=-1, keepdims=True)
        p = jnp.exp(s - m)
        p = p / jnp.sum(p, axis=-1, keepdims=True)
        o_scratch[sl, :] = jnp.dot(p, vb, preferred_element_type=jnp.float32)

    contrib = jnp.dot(o_scratch[...], wo_ref[...],
                      preferred_element_type=jnp.float32)

    @pl.when(h == 0)
    def _():
        out_ref[...] = jnp.zeros_like(out_ref)

    out_ref[...] += contrib


def allgather_call(x2d, Wdkv, Wuk, Wuv, Wkr):
    return pl.pallas_call(
        _allgather_body,
        out_shape=(
            jax.ShapeDtypeStruct((Y, BS, DC_SH), jnp.float32),
            jax.ShapeDtypeStruct((Y, DC_SH, D), jnp.float32),
            jax.ShapeDtypeStruct((Y, DC_SH, D), jnp.float32),
            jax.ShapeDtypeStruct((BS, Dr), jnp.float32),
        ),
        in_specs=[pl.BlockSpec(memory_space=pltpu.VMEM)] * 5,
        out_specs=[pl.BlockSpec(memory_space=pltpu.VMEM)] * 4,
        scratch_shapes=[
            pltpu.SemaphoreType.DMA((3, Y - 1, 2)),
            pltpu.SemaphoreType.DMA((3, Y - 1, 2)),
        ],
        compiler_params=_CompilerParams(
            vmem_limit_bytes=100 * 1024 * 1024,
        ),
    )(x2d, Wdkv, Wuk, Wuv, Wkr)


def mla_call(x2d, c_all, uk_all, uv_all, kr, Wq, Wqr_t, Wo):
    return pl.pallas_call(
        _mla_body,
        grid=(H,),
        out_shape=jax.ShapeDtypeStruct((BS, D), jnp.float32),
        in_specs=[
            pl.BlockSpec((BS, D), lambda h: (0, 0)),
            pl.BlockSpec((Y, BS, DC_SH), lambda h: (0, 0, 0)),
            pl.BlockSpec((Y, DC_SH, Dh), lambda h: (0, 0, h)),
            pl.BlockSpec((Y, DC_SH, Dh), lambda h: (0, 0, h)),
            pl.BlockSpec((BS, Dr), lambda h: (0, 0)),
            pl.BlockSpec((D, Dh), lambda h: (0, h)),
            pl.BlockSpec((1, D, Dr), lambda h: (h, 0, 0)),
            pl.BlockSpec((Dh, D), lambda h: (h, 0)),
        ],
        out_specs=pl.BlockSpec((BS, D), lambda h: (0, 0)),
        scratch_shapes=[pltpu.VMEM((BS, Dh), jnp.float32)],
        compiler_params=_CompilerParams(
            vmem_limit_bytes=100 * 1024 * 1024
        ),
    )(x2d, c_all, uk_all, uv_all, kr, Wq, Wqr_t, Wo)


def kernel(x, Wdkv, Wuk, Wuv, Wq, Wqr, Wkr, Wo):
    x2d = x.reshape(BS, D)
    c_all, uk_all, uv_all, kr = allgather_call(x2d, Wdkv, Wuk, Wuv, Wkr)
    out2d = mla_call(x2d, c_all, uk_all, uv_all, kr, Wq,
                     Wqr.reshape(D, H, Dr).transpose(1, 0, 2), Wo)
    return out2d.reshape(B, S, D)
